# baseline (speedup 1.0000x reference)
import jax
import jax.numpy as jnp
from jax import lax
from jax.experimental import pallas as pl
from jax.experimental.pallas import tpu as pltpu
from jax.experimental.pallas import tpu_sc as plsc

_NCHAN = 69
_H, _W = 720, 361
_RCHUNK = 80
_NSPLIT = _H // _RCHUNK
_NITEMS = _NCHAN * _NSPLIT
_NW = 32
_STEPS = -(-_NITEMS // _NW)
_DEPTH = 4


def _src_channel(c):
    return jnp.where(
        c < 39, c,
        jnp.where(
            c == 39, 65,
            jnp.where(
                c == 40, 66,
                jnp.where(
                    c <= 53, c - 2,
                    jnp.where(c == 54, 67, jnp.where(c <= 67, c - 3, 68))))))


def _body(x_ref, out_ref, buf0, buf1, buf2, buf3, gsem, ssem):
    wid = lax.axis_index("s") * 2 + lax.axis_index("c")
    bufs = (buf0, buf1, buf2, buf3)

    def item(i):
        t = wid + _NW * i
        return t, lax.div(t, _NSPLIT), lax.rem(t, _NSPLIT)

    def start_g(i):
        t, c, j = item(i)

        @pl.when(t < _NITEMS)
        def _():
            pltpu.async_copy(
                x_ref.at[_src_channel(c), pl.ds(j * _RCHUNK, _RCHUNK)],
                bufs[i % _DEPTH], gsem.at[i % _DEPTH])

    def wait_g(i):
        t, _, _ = item(i)

        @pl.when(t < _NITEMS)
        def _():
            pltpu.make_async_copy(
                x_ref.at[0, pl.ds(0, _RCHUNK)], bufs[i % _DEPTH],
                gsem.at[i % _DEPTH]).wait()

    def start_s(i):
        t, c, j = item(i)

        @pl.when(t < _NITEMS)
        def _():
            pltpu.async_copy(
                bufs[i % _DEPTH], out_ref.at[c, pl.ds(j * _RCHUNK, _RCHUNK)],
                ssem.at[i % _DEPTH])

    def wait_s(i):
        t, _, _ = item(i)

        @pl.when(t < _NITEMS)
        def _():
            pltpu.make_async_copy(
                bufs[i % _DEPTH], out_ref.at[0, pl.ds(0, _RCHUNK)],
                ssem.at[i % _DEPTH]).wait()

    start_g(0)
    start_g(1)
    start_g(2)
    for i in range(_STEPS):
        wait_g(i)
        start_s(i)
        if i + 3 < _STEPS:
            if i >= 1:
                wait_s(i - 1)
            start_g(i + 3)
    for i in range(max(0, _STEPS - 4), _STEPS):
        wait_s(i)


def kernel(x):
    xt = jnp.swapaxes(x, 1, 2)
    outt = pl.kernel(
        _body,
        out_type=jax.ShapeDtypeStruct((_NCHAN, _H, _W), jnp.float32),
        mesh=plsc.VectorSubcoreMesh(core_axis_name="c", subcore_axis_name="s"),
        scratch_types=[
            pltpu.VMEM((_RCHUNK, _W), jnp.float32),
            pltpu.VMEM((_RCHUNK, _W), jnp.float32),
            pltpu.VMEM((_RCHUNK, _W), jnp.float32),
            pltpu.VMEM((_RCHUNK, _W), jnp.float32),
            pltpu.SemaphoreType.DMA((4,)),
            pltpu.SemaphoreType.DMA((4,)),
        ],
    )(xt)
    return jnp.swapaxes(outt, 1, 2)


# speedup vs baseline: 1.0213x; 1.0213x over previous
import jax
import jax.numpy as jnp
from jax import lax
from jax.experimental import pallas as pl
from jax.experimental.pallas import tpu as pltpu
from jax.experimental.pallas import tpu_sc as plsc

_NCHAN = 69
_H, _W = 720, 361
_NWORK = 3
_STEPS = 12


def _src_channel(c):
    return jnp.where(
        c < 39, c,
        jnp.where(
            c == 39, 65,
            jnp.where(
                c == 40, 66,
                jnp.where(
                    c <= 53, c - 2,
                    jnp.where(c == 54, 67, jnp.where(c <= 67, c - 3, 68))))))


def _body(x_ref, out_ref, spm, gsem, ssem):
    cid = lax.axis_index("c")
    sid = lax.axis_index("s")
    nloc = 35 - cid

    @pl.when(sid < _NWORK)
    def _work():
        def chan(k):
            l = sid + _NWORK * k
            return l, cid + 2 * l

        def start_g(k):
            l, c = chan(k)

            @pl.when(l < nloc)
            def _():
                pltpu.async_copy(
                    x_ref.at[_src_channel(c)], spm.at[2 * sid + k % 2],
                    gsem.at[k % 2])

        def wait_g(k):
            l, _ = chan(k)

            @pl.when(l < nloc)
            def _():
                pltpu.make_async_copy(
                    x_ref.at[0], spm.at[2 * sid + k % 2],
                    gsem.at[k % 2]).wait()

        def start_s(k):
            l, c = chan(k)

            @pl.when(l < nloc)
            def _():
                pltpu.async_copy(
                    spm.at[2 * sid + k % 2], out_ref.at[c], ssem.at[k % 2])

        def wait_s(k):
            l, _ = chan(k)

            @pl.when(l < nloc)
            def _():
                pltpu.make_async_copy(
                    spm.at[2 * sid + k % 2], out_ref.at[0],
                    ssem.at[k % 2]).wait()

        start_g(0)
        for k in range(_STEPS):
            wait_g(k)
            start_s(k)
            if k + 1 < _STEPS:
                if k >= 1:
                    wait_s(k - 1)
                start_g(k + 1)
        wait_s(_STEPS - 2)
        wait_s(_STEPS - 1)


def kernel(x):
    xt = jnp.swapaxes(x, 1, 2)
    outt = pl.kernel(
        _body,
        out_type=jax.ShapeDtypeStruct((_NCHAN, _H, _W), jnp.float32),
        mesh=plsc.VectorSubcoreMesh(core_axis_name="c", subcore_axis_name="s"),
        scratch_types=[
            pltpu.VMEM_SHARED((2 * _NWORK, _H, _W), jnp.float32),
            pltpu.SemaphoreType.DMA((2,)),
            pltpu.SemaphoreType.DMA((2,)),
        ],
    )(xt)
    return jnp.swapaxes(outt, 1, 2)
